# baseline (speedup 1.0000x reference)
import functools

import jax
import jax.numpy as jnp
from jax import lax
from jax.experimental import pallas as pl
from jax.experimental.pallas import tpu as pltpu
from jax.experimental.pallas import tpu_sc as plsc

_LANES = 16
_NBUF = 2


@functools.lru_cache(maxsize=None)
def _build(n: int, chv: int, nw: int, unroll: int, c0: int, c1: int):
    che = chv * _LANES
    assert (n // che) * che == n
    nch = c1 - c0
    nout = nch * che
    iters = (nch + nw - 1) // nw
    outer_iters = (iters + _NBUF - 1) // _NBUF

    mesh = plsc.VectorSubcoreMesh(core_axis_name="c", subcore_axis_name="s")
    nc = 2

    @functools.partial(
        pl.kernel,
        out_type=(jax.ShapeDtypeStruct((nout,), jnp.float32),
                  jax.ShapeDtypeStruct((nout,), jnp.float32)),
        mesh=mesh,
        compiler_params=pltpu.CompilerParams(needs_layout_passes=False),
        scratch_types=[
            pltpu.VMEM((64,), jnp.float32),
            pltpu.VMEM((64,), jnp.float32),
        ] + [pltpu.VMEM((che,), jnp.int32) for _ in range(_NBUF)]
          + [pltpu.VMEM((che,), jnp.float32) for _ in range(_NBUF)]
          + [pltpu.VMEM((che,), jnp.float32) for _ in range(_NBUF)]
          + [pltpu.SemaphoreType.DMA for _ in range(2 * _NBUF)],
    )
    def k(sp_hbm, scales_hbm, shifts_hbm, s_hbm, b_hbm,
          scales_v, shifts_v, sp0, sp1, sb0, sb1, bb0, bb1,
          isem0, isem1, osem0, osem1):
        sp_bufs = [sp0, sp1]
        s_bufs = [sb0, sb1]
        b_bufs = [bb0, bb1]
        in_sems = [isem0, isem1]
        out_sems = [osem0, osem1]

        w = lax.axis_index("s") * nc + lax.axis_index("c")
        pltpu.sync_copy(scales_hbm, scales_v)
        pltpu.sync_copy(shifts_hbm, shifts_v)

        def start_in(k_, slot):
            ci = w + k_ * nw

            @pl.when(ci < nch)
            def _():
                pltpu.async_copy(sp_hbm.at[pl.ds((c0 + ci) * che, che)],
                                 sp_bufs[slot], in_sems[slot])

        def step(k_, slot):
            ci = w + k_ * nw

            @pl.when(ci < nch)
            def _():
                base = ci * che
                pltpu.make_async_copy(sp_hbm.at[pl.ds(base, che)],
                                      sp_bufs[slot], in_sems[slot]).wait()
                @pl.when(k_ >= _NBUF)
                def _():
                    pltpu.make_async_copy(s_bufs[slot],
                                          s_hbm.at[pl.ds(base, che)],
                                          out_sems[slot]).wait()
                    pltpu.make_async_copy(b_bufs[slot],
                                          b_hbm.at[pl.ds(base, che)],
                                          out_sems[slot]).wait()

                sp_b, s_b, b_b = sp_bufs[slot], s_bufs[slot], b_bufs[slot]

                @plsc.parallel_loop(0, chv, unroll=unroll)
                def _(i):
                    off = i * _LANES
                    idx = sp_b[pl.ds(off, _LANES)]
                    s_b[pl.ds(off, _LANES)] = plsc.load_gather(scales_v, [idx])
                    b_b[pl.ds(off, _LANES)] = plsc.load_gather(shifts_v, [idx])

                pltpu.async_copy(s_b, s_hbm.at[pl.ds(base, che)],
                                 out_sems[slot])
                pltpu.async_copy(b_b, b_hbm.at[pl.ds(base, che)],
                                 out_sems[slot])
                start_in(k_ + _NBUF, slot)

        for s in range(_NBUF):
            start_in(s, s)

        def outer(kk, carry):
            for s in range(_NBUF):
                step(kk * _NBUF + s, s)
            return carry

        lax.fori_loop(0, outer_iters, outer, 0)

        my_iters = (nch - w + nw - 1) // nw

        for s in range(_NBUF):
            for d in range(1, _NBUF + 1):
                k_ = my_iters - d

                @pl.when((k_ >= 0) & (k_ % _NBUF == s))
                def _():
                    pltpu.make_async_copy(s_bufs[s],
                                          s_hbm.at[pl.ds(0, che)],
                                          out_sems[s]).wait()
                    pltpu.make_async_copy(b_bufs[s],
                                          b_hbm.at[pl.ds(0, che)],
                                          out_sems[s]).wait()

    return k


def kernel(x, species, scales, shifts):
    n = x.shape[0]
    chv = 1000
    nch = n // (chv * _LANES)
    k = _build(n, chv, 32, 16, 0, nch)
    s_arr, b_arr = k(species, scales, shifts)
    return s_arr.reshape(n, 1) * x + b_arr.reshape(n, 1)


# speedup vs baseline: 1.0049x; 1.0049x over previous
import functools

import jax
import jax.numpy as jnp
from jax import lax
from jax.experimental import pallas as pl
from jax.experimental.pallas import tpu as pltpu
from jax.experimental.pallas import tpu_sc as plsc

_LANES = 16
_NBUF = 2


@functools.lru_cache(maxsize=None)
def _build(n: int, chv: int, nw: int, unroll: int, c0: int, c1: int):
    che = chv * _LANES
    assert (n // che) * che == n
    nch = c1 - c0
    nout = nch * che
    iters = (nch + nw - 1) // nw
    outer_iters = (iters + _NBUF - 1) // _NBUF

    mesh = plsc.VectorSubcoreMesh(core_axis_name="c", subcore_axis_name="s")
    nc = 2

    @functools.partial(
        pl.kernel,
        out_type=(jax.ShapeDtypeStruct((nout,), jnp.float32),
                  jax.ShapeDtypeStruct((nout,), jnp.float32)),
        mesh=mesh,
        compiler_params=pltpu.CompilerParams(needs_layout_passes=False),
        scratch_types=[
            pltpu.VMEM((64,), jnp.float32),
            pltpu.VMEM((64,), jnp.float32),
        ] + [pltpu.VMEM((che,), jnp.int32) for _ in range(_NBUF)]
          + [pltpu.VMEM((che,), jnp.float32) for _ in range(_NBUF)]
          + [pltpu.VMEM((che,), jnp.float32) for _ in range(_NBUF)]
          + [pltpu.SemaphoreType.DMA for _ in range(2 * _NBUF)],
    )
    def k(sp_hbm, scales_hbm, shifts_hbm, s_hbm, b_hbm,
          scales_v, shifts_v, sp0, sp1, sb0, sb1, bb0, bb1,
          isem0, isem1, osem0, osem1):
        sp_bufs = [sp0, sp1]
        s_bufs = [sb0, sb1]
        b_bufs = [bb0, bb1]
        in_sems = [isem0, isem1]
        out_sems = [osem0, osem1]

        w = lax.axis_index("s") * nc + lax.axis_index("c")
        pltpu.sync_copy(scales_hbm, scales_v)
        pltpu.sync_copy(shifts_hbm, shifts_v)

        def start_in(k_, slot):
            ci = w + k_ * nw

            @pl.when(ci < nch)
            def _():
                pltpu.async_copy(sp_hbm.at[pl.ds((c0 + ci) * che, che)],
                                 sp_bufs[slot], in_sems[slot])

        def step(k_, slot):
            ci = w + k_ * nw

            @pl.when(ci < nch)
            def _():
                base = ci * che
                pltpu.make_async_copy(sp_hbm.at[pl.ds(base, che)],
                                      sp_bufs[slot], in_sems[slot]).wait()
                @pl.when(k_ >= _NBUF)
                def _():
                    pltpu.make_async_copy(s_bufs[slot],
                                          s_hbm.at[pl.ds(base, che)],
                                          out_sems[slot]).wait()
                    pltpu.make_async_copy(b_bufs[slot],
                                          b_hbm.at[pl.ds(base, che)],
                                          out_sems[slot]).wait()

                sp_b, s_b, b_b = sp_bufs[slot], s_bufs[slot], b_bufs[slot]

                @plsc.parallel_loop(0, chv, unroll=unroll)
                def _(i):
                    off = i * _LANES
                    idx = sp_b[pl.ds(off, _LANES)]
                    s_b[pl.ds(off, _LANES)] = plsc.load_gather(scales_v, [idx])
                    b_b[pl.ds(off, _LANES)] = plsc.load_gather(shifts_v, [idx])

                pltpu.async_copy(s_b, s_hbm.at[pl.ds(base, che)],
                                 out_sems[slot])
                pltpu.async_copy(b_b, b_hbm.at[pl.ds(base, che)],
                                 out_sems[slot])
                start_in(k_ + _NBUF, slot)

        for s in range(_NBUF):
            start_in(s, s)

        def outer(kk, carry):
            for s in range(_NBUF):
                step(kk * _NBUF + s, s)
            return carry

        lax.fori_loop(0, outer_iters, outer, 0)

        my_iters = (nch - w + nw - 1) // nw

        for s in range(_NBUF):
            for d in range(1, _NBUF + 1):
                k_ = my_iters - d

                @pl.when((k_ >= 0) & (k_ % _NBUF == s))
                def _():
                    pltpu.make_async_copy(s_bufs[s],
                                          s_hbm.at[pl.ds(0, che)],
                                          out_sems[s]).wait()
                    pltpu.make_async_copy(b_bufs[s],
                                          b_hbm.at[pl.ds(0, che)],
                                          out_sems[s]).wait()

    return k


def kernel(x, species, scales, shifts):
    n = x.shape[0]
    chv = 1000
    nch = n // (chv * _LANES)
    k = _build(n, chv, 32, 8, 0, nch)
    s_arr, b_arr = k(species, scales, shifts)
    return s_arr.reshape(n, 1) * x + b_arr.reshape(n, 1)
